# baseline (speedup 1.0000x reference)
import functools

import jax
import jax.numpy as jnp
from jax import lax
from jax.experimental import pallas as pl
from jax.experimental.pallas import tpu as pltpu
from jax.experimental.pallas import tpu_sc as plsc

N_NODES = 10000
IN_DIM = 128
HID_DIM = 32
LAT_DIM = 16

NC = 2
NS = 16
NW = NC * NS
N_EDGES = 320000
EW = N_EDGES // NW
CHUNK = 125
NCHUNK = 80
N_PAD = 10240
ROWS_PER_TILE = N_PAD // NS

_MESH = plsc.VectorSubcoreMesh(core_axis_name="c", subcore_axis_name="s")
_SC_PARAMS = pltpu.CompilerParams(use_tc_tiling_on_sc=False)


def _deg_body(dst3_hbm, ones_hbm, zeros_hbm, out_hbm, didx_v, ones_v, acc_sp,
              ssems):
    c = lax.axis_index("c")
    s = lax.axis_index("s")
    w = c * NS + s
    pltpu.sync_copy(dst3_hbm.at[w], didx_v)
    pltpu.sync_copy(ones_hbm, ones_v)

    @pl.when(s == 0)
    def _():
        pltpu.sync_copy(zeros_hbm.at[pl.ds(0, N_NODES)], acc_sp.at[pl.ds(0, N_NODES)])
        pltpu.sync_copy(zeros_hbm.at[pl.ds(0, N_PAD - N_NODES)],
                        acc_sp.at[pl.ds(N_NODES, N_PAD - N_NODES)])

    plsc.subcore_barrier()

    def body(i, carry):
        for b in range(4):
            g = 4 * i + b

            @pl.when(g >= 4)
            def _():
                pltpu.make_async_copy(
                    ones_v, acc_sp.at[didx_v.at[0]], ssems.at[b]).wait()

            pltpu.async_copy(ones_v, acc_sp.at[didx_v.at[g]], ssems.at[b],
                             add=True)
        return carry

    lax.fori_loop(0, NCHUNK // 4, body, 0)
    for b in range(4):
        pltpu.make_async_copy(ones_v, acc_sp.at[didx_v.at[0]], ssems.at[b]).wait()
    plsc.subcore_barrier()
    pltpu.sync_copy(acc_sp.at[pl.ds(s * ROWS_PER_TILE, ROWS_PER_TILE)],
                    out_hbm.at[c, pl.ds(s * ROWS_PER_TILE, ROWS_PER_TILE)])


@functools.partial(
    pl.kernel,
    out_type=jax.ShapeDtypeStruct((NC, N_PAD, 8), jnp.float32),
    mesh=_MESH,
    scratch_types=[
        pltpu.VMEM((NCHUNK, CHUNK), jnp.int32),
        pltpu.VMEM((CHUNK, 8), jnp.float32),
        pltpu.VMEM_SHARED((N_PAD, 8), jnp.float32),
        pltpu.SemaphoreType.DMA((4,)),
    ],
    compiler_params=_SC_PARAMS,
)
def _degree_sc(dst3_hbm, ones_hbm, zeros_hbm, out_hbm, didx_v, ones_v, acc_sp,
               ssems):
    _deg_body(dst3_hbm, ones_hbm, zeros_hbm, out_hbm, didx_v, ones_v, acc_sp,
              ssems)


def _agg_body(wh_hbm, src3_hbm, dst3_hbm, zeros_hbm, out_hbm,
              sidx_v, didx_v, rows0_v, rows1_v, rows2_v, rows3_v, acc_sp,
              wh_sp, gsems, ssems):
    c = lax.axis_index("c")
    s = lax.axis_index("s")
    w = c * NS + s
    pltpu.sync_copy(src3_hbm.at[w], sidx_v)
    pltpu.sync_copy(dst3_hbm.at[w], didx_v)

    @pl.when(s == 0)
    def _():
        @pl.when(c == 0)
        def _():
            pltpu.sync_copy(wh_hbm.at[pl.ds(0, N_NODES)], acc_sp.at[pl.ds(0, N_NODES)])

        @pl.when(c != 0)
        def _():
            pltpu.sync_copy(zeros_hbm.at[pl.ds(0, N_NODES)], acc_sp.at[pl.ds(0, N_NODES)])

        pltpu.sync_copy(zeros_hbm.at[pl.ds(0, N_PAD - N_NODES)],
                        acc_sp.at[pl.ds(N_NODES, N_PAD - N_NODES)])

    @pl.when(s == 1)
    def _():
        pltpu.sync_copy(wh_hbm.at[pl.ds(0, N_NODES)], wh_sp.at[pl.ds(0, N_NODES)])

    plsc.subcore_barrier()

    rows = [rows0_v, rows1_v, rows2_v, rows3_v]
    pltpu.async_copy(wh_sp.at[sidx_v.at[0]], rows[0], gsems.at[0])
    pltpu.async_copy(wh_sp.at[sidx_v.at[1]], rows[1], gsems.at[1])

    def body(i, carry):
        for b in range(4):
            g = 4 * i + b
            b2 = (b + 2) % 4

            @pl.when(g >= 2)
            def _():
                pltpu.make_async_copy(
                    rows[b2], acc_sp.at[didx_v.at[0]], ssems.at[b2]).wait()

            @pl.when(g + 2 < NCHUNK)
            def _():
                pltpu.async_copy(wh_sp.at[sidx_v.at[g + 2]], rows[b2],
                                 gsems.at[b2])

            pltpu.make_async_copy(wh_sp.at[sidx_v.at[g]], rows[b],
                                  gsems.at[b]).wait()
            pltpu.async_copy(rows[b], acc_sp.at[didx_v.at[g]], ssems.at[b],
                             add=True)
        return carry

    lax.fori_loop(0, NCHUNK // 4, body, 0)
    pltpu.make_async_copy(rows[(NCHUNK - 2) % 4], acc_sp.at[didx_v.at[0]],
                          ssems.at[(NCHUNK - 2) % 4]).wait()
    pltpu.make_async_copy(rows[(NCHUNK - 1) % 4], acc_sp.at[didx_v.at[0]],
                          ssems.at[(NCHUNK - 1) % 4]).wait()
    plsc.subcore_barrier()
    pltpu.sync_copy(acc_sp.at[pl.ds(s * ROWS_PER_TILE, ROWS_PER_TILE)],
                    out_hbm.at[c, pl.ds(s * ROWS_PER_TILE, ROWS_PER_TILE)])


def _make_agg(d):
    @functools.partial(
        pl.kernel,
        out_type=jax.ShapeDtypeStruct((NC, N_PAD, d), jnp.float32),
        mesh=_MESH,
        scratch_types=[
            pltpu.VMEM((NCHUNK, CHUNK), jnp.int32),
            pltpu.VMEM((NCHUNK, CHUNK), jnp.int32),
            pltpu.VMEM((CHUNK, d), jnp.float32),
            pltpu.VMEM((CHUNK, d), jnp.float32),
            pltpu.VMEM((CHUNK, d), jnp.float32),
            pltpu.VMEM((CHUNK, d), jnp.float32),
            pltpu.VMEM_SHARED((N_PAD, d), jnp.float32),
            pltpu.VMEM_SHARED((N_PAD, d), jnp.float32),
            pltpu.SemaphoreType.DMA((4,)),
            pltpu.SemaphoreType.DMA((4,)),
        ],
        compiler_params=_SC_PARAMS,
    )
    def agg(wh_hbm, src3_hbm, dst3_hbm, zeros_hbm, out_hbm,
            sidx_v, didx_v, rows0_v, rows1_v, rows2_v, rows3_v, acc_sp,
            wh_sp, gsems, ssems):
        _agg_body(wh_hbm, src3_hbm, dst3_hbm, zeros_hbm, out_hbm,
                  sidx_v, didx_v, rows0_v, rows1_v, rows2_v, rows3_v, acc_sp,
                  wh_sp, gsems, ssems)

    return agg


_agg32 = _make_agg(HID_DIM)
_agg16 = _make_agg(LAT_DIM)


def _mm_body(x_ref, w_ref, out_ref):
    out_ref[...] = jnp.dot(x_ref[...], w_ref[...],
                           preferred_element_type=jnp.float32)


def _mm(x, w, bm=1000):
    n, k = x.shape
    d = w.shape[1]
    return pl.pallas_call(
        _mm_body,
        grid=(n // bm,),
        in_specs=[
            pl.BlockSpec((bm, k), lambda i: (i, 0)),
            pl.BlockSpec((k, d), lambda i: (0, 0)),
        ],
        out_specs=pl.BlockSpec((bm, d), lambda i: (i, 0)),
        out_shape=jax.ShapeDtypeStruct((n, d), jnp.float32),
    )(x, w)


def _scale_body(h_ref, degp_ref, out_ref, deg_ref):
    dp = degp_ref[...]
    deg = dp[0, :, 0:1] + dp[1, :, 0:1] + 1.0
    deg_ref[...] = deg
    out_ref[...] = h_ref[...] * jax.lax.rsqrt(deg)


def _scale(h, degparts, bm=1000):
    n, d = h.shape
    return pl.pallas_call(
        _scale_body,
        grid=(n // bm,),
        in_specs=[
            pl.BlockSpec((bm, d), lambda i: (i, 0)),
            pl.BlockSpec((NC, bm, 8), lambda i: (0, i, 0)),
        ],
        out_specs=[
            pl.BlockSpec((bm, d), lambda i: (i, 0)),
            pl.BlockSpec((bm, 1), lambda i: (i, 0)),
        ],
        out_shape=[
            jax.ShapeDtypeStruct((n, d), jnp.float32),
            jax.ShapeDtypeStruct((n, 1), jnp.float32),
        ],
    )(h, degparts)


def _post_mm_body(accp_ref, deg_ref, b_ref, w_ref, out_ref):
    ap = accp_ref[...]
    dis = jax.lax.rsqrt(deg_ref[...])
    h = jnp.maximum((ap[0] + ap[1]) * dis + b_ref[...], 0.0)
    out_ref[...] = jnp.dot(h, w_ref[...], preferred_element_type=jnp.float32) * dis


def _post_mm(accparts, deg2d, b, w, bm=1000):
    n = deg2d.shape[0]
    k = accparts.shape[2]
    d = w.shape[1]
    return pl.pallas_call(
        _post_mm_body,
        grid=(n // bm,),
        in_specs=[
            pl.BlockSpec((NC, bm, k), lambda i: (0, i, 0)),
            pl.BlockSpec((bm, 1), lambda i: (i, 0)),
            pl.BlockSpec((1, k), lambda i: (0, 0)),
            pl.BlockSpec((k, d), lambda i: (0, 0)),
        ],
        out_specs=pl.BlockSpec((bm, d), lambda i: (i, 0)),
        out_shape=jax.ShapeDtypeStruct((n, d), jnp.float32),
    )(accparts, deg2d, b.reshape(1, k), w)


def _final_body(accp_ref, deg_ref, b_ref, out_ref):
    ap = accp_ref[...]
    out_ref[...] = (ap[0] + ap[1]) * jax.lax.rsqrt(deg_ref[...]) + b_ref[...]


def _final(accparts, deg2d, b, bm=1000):
    n = deg2d.shape[0]
    d = accparts.shape[2]
    return pl.pallas_call(
        _final_body,
        grid=(n // bm,),
        in_specs=[
            pl.BlockSpec((NC, bm, d), lambda i: (0, i, 0)),
            pl.BlockSpec((bm, 1), lambda i: (i, 0)),
            pl.BlockSpec((1, d), lambda i: (0, 0)),
        ],
        out_specs=pl.BlockSpec((bm, d), lambda i: (i, 0)),
        out_shape=jax.ShapeDtypeStruct((n, d), jnp.float32),
    )(accparts, deg2d, b.reshape(1, d))


def _decoder_body(zi_ref, zj_ref, out_ref):
    g = jax.lax.dot_general(
        zi_ref[...], zj_ref[...], (((1,), (1,)), ((), ())),
        preferred_element_type=jnp.float32,
        precision=jax.lax.Precision.DEFAULT,
    )
    out_ref[...] = jax.nn.sigmoid(g)


def _decoder(z, bm=1024, bn=2048):
    n, d = z.shape
    grid = (pl.cdiv(n, bm), pl.cdiv(n, bn))
    return pl.pallas_call(
        _decoder_body,
        grid=grid,
        in_specs=[
            pl.BlockSpec((bm, d), lambda i, j: (i, 0)),
            pl.BlockSpec((bn, d), lambda i, j: (j, 0)),
        ],
        out_specs=pl.BlockSpec((bm, bn), lambda i, j: (i, j)),
        out_shape=jax.ShapeDtypeStruct((n, n), jnp.float32),
    )(z, z)


def kernel(x, edge_index, W1, b1, W2, b2):
    n = x.shape[0]
    src3 = edge_index[0].astype(jnp.int32).reshape(NW, NCHUNK, CHUNK)
    dst3 = edge_index[1].astype(jnp.int32).reshape(NW, NCHUNK, CHUNK)

    ones128 = jnp.ones((CHUNK, 8), jnp.float32)
    zeros8 = jnp.zeros((n, 8), jnp.float32)
    zeros16 = jnp.zeros((n, 16), jnp.float32)
    zeros32 = jnp.zeros((n, HID_DIM), jnp.float32)

    h1 = _mm(x, W1)
    degparts = _degree_sc(dst3, ones128, zeros8)
    wh1, deg2d = _scale(h1, degparts)
    acc1parts = _agg32(wh1, src3, dst3, zeros32)
    wh2 = _post_mm(acc1parts, deg2d, b1, W2)
    acc2parts = _agg16(wh2, src3, dst3, zeros16)
    z = _final(acc2parts, deg2d, b2)
    adj_pred = _decoder(z)
    return (z, adj_pred)


# speedup vs baseline: 1.0063x; 1.0063x over previous
import functools

import jax
import jax.numpy as jnp
from jax import lax
from jax.experimental import pallas as pl
from jax.experimental.pallas import tpu as pltpu
from jax.experimental.pallas import tpu_sc as plsc

N_NODES = 10000
IN_DIM = 128
HID_DIM = 32
LAT_DIM = 16

NC = 2
NS = 16
NW = NC * NS
N_EDGES = 320000
EW = N_EDGES // NW
CHUNK = 125
NCHUNK = 80
N_PAD = 10240
ROWS_PER_TILE = N_PAD // NS

_MESH = plsc.VectorSubcoreMesh(core_axis_name="c", subcore_axis_name="s")
_SC_PARAMS = pltpu.CompilerParams(use_tc_tiling_on_sc=False)


def _deg_body(dst3_hbm, ones_hbm, zeros_hbm, out_hbm, didx_v, ones_v, acc_sp,
              ssems):
    c = lax.axis_index("c")
    s = lax.axis_index("s")
    w = c * NS + s
    pltpu.sync_copy(dst3_hbm.at[w], didx_v)
    pltpu.sync_copy(ones_hbm, ones_v)

    @pl.when(s == 0)
    def _():
        pltpu.sync_copy(zeros_hbm.at[pl.ds(0, N_NODES)], acc_sp.at[pl.ds(0, N_NODES)])
        pltpu.sync_copy(zeros_hbm.at[pl.ds(0, N_PAD - N_NODES)],
                        acc_sp.at[pl.ds(N_NODES, N_PAD - N_NODES)])

    plsc.subcore_barrier()

    def body(i, carry):
        for b in range(4):
            g = 4 * i + b

            @pl.when(g >= 4)
            def _():
                pltpu.make_async_copy(
                    ones_v, acc_sp.at[didx_v.at[0]], ssems.at[b]).wait()

            pltpu.async_copy(ones_v, acc_sp.at[didx_v.at[g]], ssems.at[b],
                             add=True)
        return carry

    lax.fori_loop(0, NCHUNK // 4, body, 0)
    for b in range(4):
        pltpu.make_async_copy(ones_v, acc_sp.at[didx_v.at[0]], ssems.at[b]).wait()
    plsc.subcore_barrier()
    pltpu.sync_copy(acc_sp.at[pl.ds(s * ROWS_PER_TILE, ROWS_PER_TILE)],
                    out_hbm.at[c, pl.ds(s * ROWS_PER_TILE, ROWS_PER_TILE)])


@functools.partial(
    pl.kernel,
    out_type=jax.ShapeDtypeStruct((NC, N_PAD, 16), jnp.float32),
    mesh=_MESH,
    scratch_types=[
        pltpu.VMEM((NCHUNK, CHUNK), jnp.int32),
        pltpu.VMEM((CHUNK, 16), jnp.float32),
        pltpu.VMEM_SHARED((N_PAD, 16), jnp.float32),
        pltpu.SemaphoreType.DMA((4,)),
    ],
    compiler_params=_SC_PARAMS,
)
def _degree_sc(dst3_hbm, ones_hbm, zeros_hbm, out_hbm, didx_v, ones_v, acc_sp,
               ssems):
    _deg_body(dst3_hbm, ones_hbm, zeros_hbm, out_hbm, didx_v, ones_v, acc_sp,
              ssems)


def _agg_body(wh_hbm, src3_hbm, dst3_hbm, zeros_hbm, out_hbm,
              sidx_v, didx_v, rows0_v, rows1_v, rows2_v, rows3_v, acc_sp,
              wh_sp, gsems, ssems):
    c = lax.axis_index("c")
    s = lax.axis_index("s")
    w = c * NS + s
    pltpu.sync_copy(src3_hbm.at[w], sidx_v)
    pltpu.sync_copy(dst3_hbm.at[w], didx_v)

    @pl.when(s == 0)
    def _():
        @pl.when(c == 0)
        def _():
            pltpu.sync_copy(wh_hbm.at[pl.ds(0, N_NODES)], acc_sp.at[pl.ds(0, N_NODES)])

        @pl.when(c != 0)
        def _():
            pltpu.sync_copy(zeros_hbm.at[pl.ds(0, N_NODES)], acc_sp.at[pl.ds(0, N_NODES)])

        pltpu.sync_copy(zeros_hbm.at[pl.ds(0, N_PAD - N_NODES)],
                        acc_sp.at[pl.ds(N_NODES, N_PAD - N_NODES)])

    @pl.when(s == 1)
    def _():
        pltpu.sync_copy(wh_hbm.at[pl.ds(0, N_NODES)], wh_sp.at[pl.ds(0, N_NODES)])

    plsc.subcore_barrier()

    rows = [rows0_v, rows1_v, rows2_v, rows3_v]
    pltpu.async_copy(wh_sp.at[sidx_v.at[0]], rows[0], gsems.at[0])
    pltpu.async_copy(wh_sp.at[sidx_v.at[1]], rows[1], gsems.at[1])

    def body(i, carry):
        for b in range(4):
            g = 4 * i + b
            b2 = (b + 2) % 4

            @pl.when(g >= 2)
            def _():
                pltpu.make_async_copy(
                    rows[b2], acc_sp.at[didx_v.at[0]], ssems.at[b2]).wait()

            @pl.when(g + 2 < NCHUNK)
            def _():
                pltpu.async_copy(wh_sp.at[sidx_v.at[g + 2]], rows[b2],
                                 gsems.at[b2])

            pltpu.make_async_copy(wh_sp.at[sidx_v.at[g]], rows[b],
                                  gsems.at[b]).wait()
            pltpu.async_copy(rows[b], acc_sp.at[didx_v.at[g]], ssems.at[b],
                             add=True)
        return carry

    lax.fori_loop(0, NCHUNK // 4, body, 0)
    pltpu.make_async_copy(rows[(NCHUNK - 2) % 4], acc_sp.at[didx_v.at[0]],
                          ssems.at[(NCHUNK - 2) % 4]).wait()
    pltpu.make_async_copy(rows[(NCHUNK - 1) % 4], acc_sp.at[didx_v.at[0]],
                          ssems.at[(NCHUNK - 1) % 4]).wait()
    plsc.subcore_barrier()
    pltpu.sync_copy(acc_sp.at[pl.ds(s * ROWS_PER_TILE, ROWS_PER_TILE)],
                    out_hbm.at[c, pl.ds(s * ROWS_PER_TILE, ROWS_PER_TILE)])


def _make_agg(d):
    @functools.partial(
        pl.kernel,
        out_type=jax.ShapeDtypeStruct((NC, N_PAD, d), jnp.float32),
        mesh=_MESH,
        scratch_types=[
            pltpu.VMEM((NCHUNK, CHUNK), jnp.int32),
            pltpu.VMEM((NCHUNK, CHUNK), jnp.int32),
            pltpu.VMEM((CHUNK, d), jnp.float32),
            pltpu.VMEM((CHUNK, d), jnp.float32),
            pltpu.VMEM((CHUNK, d), jnp.float32),
            pltpu.VMEM((CHUNK, d), jnp.float32),
            pltpu.VMEM_SHARED((N_PAD, d), jnp.float32),
            pltpu.VMEM_SHARED((N_PAD, d), jnp.float32),
            pltpu.SemaphoreType.DMA((4,)),
            pltpu.SemaphoreType.DMA((4,)),
        ],
        compiler_params=_SC_PARAMS,
    )
    def agg(wh_hbm, src3_hbm, dst3_hbm, zeros_hbm, out_hbm,
            sidx_v, didx_v, rows0_v, rows1_v, rows2_v, rows3_v, acc_sp,
            wh_sp, gsems, ssems):
        _agg_body(wh_hbm, src3_hbm, dst3_hbm, zeros_hbm, out_hbm,
                  sidx_v, didx_v, rows0_v, rows1_v, rows2_v, rows3_v, acc_sp,
                  wh_sp, gsems, ssems)

    return agg


_agg32 = _make_agg(HID_DIM)
_agg16 = _make_agg(LAT_DIM)


def _mm_body(x_ref, w_ref, out_ref):
    out_ref[...] = jnp.dot(x_ref[...], w_ref[...],
                           preferred_element_type=jnp.float32)


def _mm(x, w, bm=1000):
    n, k = x.shape
    d = w.shape[1]
    return pl.pallas_call(
        _mm_body,
        grid=(n // bm,),
        in_specs=[
            pl.BlockSpec((bm, k), lambda i: (i, 0)),
            pl.BlockSpec((k, d), lambda i: (0, 0)),
        ],
        out_specs=pl.BlockSpec((bm, d), lambda i: (i, 0)),
        out_shape=jax.ShapeDtypeStruct((n, d), jnp.float32),
    )(x, w)


def _scale_body(h_ref, degp_ref, out_ref, deg_ref):
    dp = degp_ref[...]
    deg = dp[0, :, 0:1] + dp[1, :, 0:1] + 1.0
    deg_ref[...] = deg
    out_ref[...] = h_ref[...] * jax.lax.rsqrt(deg)


def _scale(h, degparts, bm=1000):
    n, d = h.shape
    return pl.pallas_call(
        _scale_body,
        grid=(n // bm,),
        in_specs=[
            pl.BlockSpec((bm, d), lambda i: (i, 0)),
            pl.BlockSpec((NC, bm, 16), lambda i: (0, i, 0)),
        ],
        out_specs=[
            pl.BlockSpec((bm, d), lambda i: (i, 0)),
            pl.BlockSpec((bm, 1), lambda i: (i, 0)),
        ],
        out_shape=[
            jax.ShapeDtypeStruct((n, d), jnp.float32),
            jax.ShapeDtypeStruct((n, 1), jnp.float32),
        ],
    )(h, degparts)


def _post_mm_body(accp_ref, deg_ref, b_ref, w_ref, out_ref):
    ap = accp_ref[...]
    dis = jax.lax.rsqrt(deg_ref[...])
    h = jnp.maximum((ap[0] + ap[1]) * dis + b_ref[...], 0.0)
    out_ref[...] = jnp.dot(h, w_ref[...], preferred_element_type=jnp.float32) * dis


def _post_mm(accparts, deg2d, b, w, bm=1000):
    n = deg2d.shape[0]
    k = accparts.shape[2]
    d = w.shape[1]
    return pl.pallas_call(
        _post_mm_body,
        grid=(n // bm,),
        in_specs=[
            pl.BlockSpec((NC, bm, k), lambda i: (0, i, 0)),
            pl.BlockSpec((bm, 1), lambda i: (i, 0)),
            pl.BlockSpec((1, k), lambda i: (0, 0)),
            pl.BlockSpec((k, d), lambda i: (0, 0)),
        ],
        out_specs=pl.BlockSpec((bm, d), lambda i: (i, 0)),
        out_shape=jax.ShapeDtypeStruct((n, d), jnp.float32),
    )(accparts, deg2d, b.reshape(1, k), w)


def _final_body(accp_ref, deg_ref, b_ref, out_ref):
    ap = accp_ref[...]
    out_ref[...] = (ap[0] + ap[1]) * jax.lax.rsqrt(deg_ref[...]) + b_ref[...]


def _final(accparts, deg2d, b, bm=1000):
    n = deg2d.shape[0]
    d = accparts.shape[2]
    return pl.pallas_call(
        _final_body,
        grid=(n // bm,),
        in_specs=[
            pl.BlockSpec((NC, bm, d), lambda i: (0, i, 0)),
            pl.BlockSpec((bm, 1), lambda i: (i, 0)),
            pl.BlockSpec((1, d), lambda i: (0, 0)),
        ],
        out_specs=pl.BlockSpec((bm, d), lambda i: (i, 0)),
        out_shape=jax.ShapeDtypeStruct((n, d), jnp.float32),
    )(accparts, deg2d, b.reshape(1, d))


def _decoder_body(zi_ref, zj_ref, out_ref):
    g = jax.lax.dot_general(
        zi_ref[...], zj_ref[...], (((1,), (1,)), ((), ())),
        preferred_element_type=jnp.float32,
        precision=jax.lax.Precision.DEFAULT,
    )
    out_ref[...] = jax.nn.sigmoid(g)


def _decoder(z, bm=1024, bn=2048):
    n, d = z.shape
    grid = (pl.cdiv(n, bm), pl.cdiv(n, bn))
    return pl.pallas_call(
        _decoder_body,
        grid=grid,
        in_specs=[
            pl.BlockSpec((bm, d), lambda i, j: (i, 0)),
            pl.BlockSpec((bn, d), lambda i, j: (j, 0)),
        ],
        out_specs=pl.BlockSpec((bm, bn), lambda i, j: (i, j)),
        out_shape=jax.ShapeDtypeStruct((n, n), jnp.float32),
    )(z, z)


def kernel(x, edge_index, W1, b1, W2, b2):
    n = x.shape[0]
    src3 = edge_index[0].astype(jnp.int32).reshape(NW, NCHUNK, CHUNK)
    dst3 = edge_index[1].astype(jnp.int32).reshape(NW, NCHUNK, CHUNK)

    ones128 = jnp.ones((CHUNK, 16), jnp.float32)
    zeros16 = jnp.zeros((n, 16), jnp.float32)
    zeros32 = jnp.zeros((n, HID_DIM), jnp.float32)

    h1 = _mm(x, W1)
    degparts = _degree_sc(dst3, ones128, zeros16)
    wh1, deg2d = _scale(h1, degparts)
    acc1parts = _agg32(wh1, src3, dst3, zeros32)
    wh2 = _post_mm(acc1parts, deg2d, b1, W2)
    acc2parts = _agg16(wh2, src3, dst3, zeros16)
    z = _final(acc2parts, deg2d, b2)
    adj_pred = _decoder(z)
    return (z, adj_pred)
